# baseline (speedup 1.0000x reference)
import jax
import jax.numpy as jnp
from jax.experimental import pallas as pl
from jax.experimental.pallas import tpu as pltpu

_TN = 512


def _tile_kernel(s_ref, sb_ref, tb_ref, t2h_ref, thl_ref, out_ref):
    s = s_ref[0]
    prod = jax.lax.dot_general(
        sb_ref[0], tb_ref[0], (((1,), (0,)), ((), ())),
        preferred_element_type=jnp.float32,
    )
    q = t2h_ref[0] - prod
    rowmin = jnp.min(q, axis=1)
    onehot = jnp.where(q == rowmin[:, None], 1.0, 0.0).astype(jnp.bfloat16)
    g = jax.lax.dot_general(
        onehot, thl_ref[0], (((1,), (0,)), ((), ())),
        preferred_element_type=jnp.float32,
    )
    tsel = (g[:, 0:3] + g[:, 3:6]) / g[:, 6:7]
    diff = s - tsel
    out_ref[0, 0] = jnp.full((8, 128), jnp.sum(diff * diff), jnp.float32)


def _bf16_hi(x):
    return x.astype(jnp.bfloat16).astype(jnp.float32)


def kernel(source_point_cloud, target_point_cloud):
    B, N, _ = source_point_cloud.shape
    M = target_point_cloud.shape[1]
    nt = N // _TN
    bf16 = jnp.bfloat16

    src = source_point_cloud
    tgt = target_point_cloud

    s_bf = src.astype(bf16)
    t_bf = jnp.transpose(tgt, (0, 2, 1)).astype(bf16)
    t2h = 0.5 * jnp.sum(tgt * tgt, axis=2)[:, None, :]

    th = _bf16_hi(tgt)
    thl = jnp.concatenate(
        [th.astype(bf16), (tgt - th).astype(bf16), jnp.ones((B, M, 1), bf16)],
        axis=2,
    )

    partials = pl.pallas_call(
        _tile_kernel,
        grid=(B, nt),
        in_specs=[
            pl.BlockSpec((1, _TN, 3), lambda b, i: (b, i, 0)),
            pl.BlockSpec((1, _TN, 3), lambda b, i: (b, i, 0)),
            pl.BlockSpec((1, 3, M), lambda b, i: (b, 0, 0)),
            pl.BlockSpec((1, 1, M), lambda b, i: (b, 0, 0)),
            pl.BlockSpec((1, M, 7), lambda b, i: (b, 0, 0)),
        ],
        out_specs=pl.BlockSpec((1, 1, 8, 128), lambda b, i: (b, i, 0, 0)),
        out_shape=jax.ShapeDtypeStruct((B, nt, 8, 128), jnp.float32),
        compiler_params=pltpu.CompilerParams(
            dimension_semantics=("parallel", "parallel"),
            vmem_limit_bytes=128 * 1024 * 1024,
        ),
    )(src, s_bf, t_bf, t2h, thl)

    return jnp.sum(partials[:, :, 0, 0]) / (B * N * 3)


# speedup vs baseline: 1.5335x; 1.5335x over previous
import jax
import jax.numpy as jnp
from jax.experimental import pallas as pl
from jax.experimental.pallas import tpu as pltpu

_TN = 256


def _tile_kernel(s_ref, sb_ref, tb_ref, t2h_ref, thl_ref, out_ref):
    s = s_ref[0]
    prod = jax.lax.dot_general(
        sb_ref[0], tb_ref[0], (((1,), (0,)), ((), ())),
        preferred_element_type=jnp.float32,
    )
    q = t2h_ref[0] - prod
    rowmin = jnp.min(q, axis=1)
    onehot = jnp.where(q == rowmin[:, None], 1.0, 0.0).astype(jnp.bfloat16)
    g = jax.lax.dot_general(
        onehot, thl_ref[0], (((1,), (0,)), ((), ())),
        preferred_element_type=jnp.float32,
    )
    tsel = (g[:, 0:3] + g[:, 3:6]) / g[:, 6:7]
    diff = s - tsel
    out_ref[0, 0] = jnp.full((8, 128), jnp.sum(diff * diff), jnp.float32)


def _bf16_hi(x):
    return x.astype(jnp.bfloat16).astype(jnp.float32)


def kernel(source_point_cloud, target_point_cloud):
    B, N, _ = source_point_cloud.shape
    M = target_point_cloud.shape[1]
    nt = N // _TN
    bf16 = jnp.bfloat16

    src = source_point_cloud
    tgt = target_point_cloud

    s_bf = src.astype(bf16)
    t_bf = jnp.transpose(tgt, (0, 2, 1)).astype(bf16)
    t2h = 0.5 * jnp.sum(tgt * tgt, axis=2)[:, None, :]

    th = _bf16_hi(tgt)
    thl = jnp.concatenate(
        [th.astype(bf16), (tgt - th).astype(bf16), jnp.ones((B, M, 1), bf16)],
        axis=2,
    )

    partials = pl.pallas_call(
        _tile_kernel,
        grid=(B, nt),
        in_specs=[
            pl.BlockSpec((1, _TN, 3), lambda b, i: (b, i, 0)),
            pl.BlockSpec((1, _TN, 3), lambda b, i: (b, i, 0)),
            pl.BlockSpec((1, 3, M), lambda b, i: (b, 0, 0)),
            pl.BlockSpec((1, 1, M), lambda b, i: (b, 0, 0)),
            pl.BlockSpec((1, M, 7), lambda b, i: (b, 0, 0)),
        ],
        out_specs=pl.BlockSpec((1, 1, 8, 128), lambda b, i: (b, i, 0, 0)),
        out_shape=jax.ShapeDtypeStruct((B, nt, 8, 128), jnp.float32),
        compiler_params=pltpu.CompilerParams(
            dimension_semantics=("parallel", "arbitrary"),
            vmem_limit_bytes=128 * 1024 * 1024,
        ),
    )(src, s_bf, t_bf, t2h, thl)

    return jnp.sum(partials[:, :, 0, 0]) / (B * N * 3)
